# baseline (speedup 1.0000x reference)
import functools

import jax
import jax.numpy as jnp
from jax.experimental import pallas as pl
from jax.experimental.pallas import tpu as pltpu

_EPS = 1e-6


def _signed_log1p(x):
    xi = jax.lax.bitcast_convert_type(x, jnp.uint32)
    sbit = xi & jnp.uint32(0x80000000)
    ax = jax.lax.bitcast_convert_type(xi & jnp.uint32(0x7FFFFFFF), jnp.float32)
    lg = jnp.log(1.0 + ax)
    li = jax.lax.bitcast_convert_type(lg, jnp.uint32)
    return jax.lax.bitcast_convert_type(li | sbit, jnp.float32)


def _body(x_ref, sl_ref, tl_ref, o_ref, acc_ref, *, nb, total_rows):
    p = pl.program_id(0)
    i = pl.program_id(1)

    @pl.when(p == 0)
    def _stats_phase():
        x = x_ref[...]
        s = jnp.sum(x, axis=0, keepdims=True)
        ss = jnp.sum(x * x, axis=0, keepdims=True)
        mn = jnp.min(x, axis=0, keepdims=True)
        mx = jnp.max(x, axis=0, keepdims=True)

        @pl.when(i == 0)
        def _init():
            acc_ref[0:1, :] = s
            acc_ref[1:2, :] = ss
            acc_ref[2:3, :] = mn
            acc_ref[3:4, :] = mx

        @pl.when(i > 0)
        def _accum():
            acc_ref[0:1, :] += s
            acc_ref[1:2, :] += ss
            acc_ref[2:3, :] = jnp.minimum(acc_ref[2:3, :], mn)
            acc_ref[3:4, :] = jnp.maximum(acc_ref[3:4, :], mx)

        @pl.when(i == nb - 1)
        def _finalize():
            tot = acc_ref[0:1, :]
            totsq = acc_ref[1:2, :]
            cmn = acc_ref[2:3, :]
            cmx = acc_ref[3:4, :]
            mean = tot / total_rows
            var = jnp.maximum(totsq / total_rows - mean * mean, 0.0)
            std = jnp.sqrt(var)
            ma = jnp.maximum(jnp.abs(cmn), jnp.abs(cmx))
            a0 = 1.0 / (std + _EPS)
            b0 = -mean * a0
            a1 = 1.0 / (cmx - cmn + _EPS)
            b1 = -cmn * a1
            a2 = 1.0 / (ma + _EPS)
            tl = tl_ref[...]
            kmax = jnp.max(tl, axis=0, keepdims=True)
            kcap = tl.shape[0]
            jidx = jax.lax.broadcasted_iota(jnp.int32, tl.shape, 0)
            kidx = jnp.min(jnp.where(tl == kmax, jidx, kcap), axis=0, keepdims=True)
            gate = sl_ref[...] > 0.0
            use_log = gate & (kidx == 3)
            affine = gate & (kidx != 3)
            a_sel = jnp.where(kidx == 0, a0, jnp.where(kidx == 1, a1, a2))
            b_sel = jnp.where(kidx == 0, b0, jnp.where(kidx == 1, b1, 0.0))
            acc_ref[4:5, :] = jnp.where(affine, a_sel, 1.0)
            acc_ref[5:6, :] = jnp.where(affine, b_sel, 0.0)
            acc_ref[6:7, :] = jnp.where(use_log, 1.0, 0.0)

    @pl.when(p == 1)
    def _apply_phase():
        a = acc_ref[4:5, :]
        b = acc_ref[5:6, :]
        use_log = acc_ref[6:7, :] > 0.5
        x = x_ref[...]
        lin = x * a + b
        o_ref[...] = jnp.where(use_log, _signed_log1p(x), lin)


def kernel(X, step_prob_logits, tf_prob_logits, is_train, max_only):
    B, F = X.shape
    K = tf_prob_logits.shape[1]
    sl = step_prob_logits.reshape(1, F)
    tl = tf_prob_logits.T
    nb = 1
    rb = B // nb

    return pl.pallas_call(
        functools.partial(_body, nb=nb, total_rows=B),
        grid=(2, nb),
        in_specs=[
            pl.BlockSpec((rb, F), lambda p, i: (i, 0)),
            pl.BlockSpec((1, F), lambda p, i: (0, 0)),
            pl.BlockSpec((K, F), lambda p, i: (0, 0)),
        ],
        out_specs=pl.BlockSpec((rb, F), lambda p, i: (p * i, 0)),
        out_shape=jax.ShapeDtypeStruct((B, F), X.dtype),
        scratch_shapes=[pltpu.VMEM((8, F), jnp.float32)],
    )(X, sl, tl)


# speedup vs baseline: 1.2147x; 1.2147x over previous
import functools

import jax
import jax.numpy as jnp
from jax.experimental import pallas as pl
from jax.experimental.pallas import tpu as pltpu

_EPS = 1e-6


def _signed_log1p(x):
    xi = jax.lax.bitcast_convert_type(x, jnp.uint32)
    sbit = xi & jnp.uint32(0x80000000)
    ax = jax.lax.bitcast_convert_type(xi & jnp.uint32(0x7FFFFFFF), jnp.float32)
    lg = jnp.log(1.0 + ax)
    li = jax.lax.bitcast_convert_type(lg, jnp.uint32)
    return jax.lax.bitcast_convert_type(li | sbit, jnp.float32)


def _body(x_ref, sl_ref, tl_ref, o_ref, acc_ref, xs_ref, *, nb, total_rows):
    p = pl.program_id(0)
    i = pl.program_id(1)
    rb = x_ref.shape[0]

    @pl.when(p == 0)
    def _stats_phase():
        x = x_ref[...]
        xs_ref[pl.ds(i * rb, rb), :] = x
        s = jnp.sum(x, axis=0, keepdims=True)
        ss = jnp.sum(x * x, axis=0, keepdims=True)
        mn = jnp.min(x, axis=0, keepdims=True)
        mx = jnp.max(x, axis=0, keepdims=True)

        @pl.when(i == 0)
        def _init():
            acc_ref[0:1, :] = s
            acc_ref[1:2, :] = ss
            acc_ref[2:3, :] = mn
            acc_ref[3:4, :] = mx

        @pl.when(i > 0)
        def _accum():
            acc_ref[0:1, :] += s
            acc_ref[1:2, :] += ss
            acc_ref[2:3, :] = jnp.minimum(acc_ref[2:3, :], mn)
            acc_ref[3:4, :] = jnp.maximum(acc_ref[3:4, :], mx)

        @pl.when(i == nb - 1)
        def _finalize():
            tot = acc_ref[0:1, :]
            totsq = acc_ref[1:2, :]
            cmn = acc_ref[2:3, :]
            cmx = acc_ref[3:4, :]
            mean = tot / total_rows
            var = jnp.maximum(totsq / total_rows - mean * mean, 0.0)
            std = jnp.sqrt(var)
            ma = jnp.maximum(jnp.abs(cmn), jnp.abs(cmx))
            a0 = 1.0 / (std + _EPS)
            b0 = -mean * a0
            a1 = 1.0 / (cmx - cmn + _EPS)
            b1 = -cmn * a1
            a2 = 1.0 / (ma + _EPS)
            tl = tl_ref[...]
            kmax = jnp.max(tl, axis=0, keepdims=True)
            kcap = tl.shape[0]
            jidx = jax.lax.broadcasted_iota(jnp.int32, tl.shape, 0)
            kidx = jnp.min(jnp.where(tl == kmax, jidx, kcap), axis=0, keepdims=True)
            gate = sl_ref[...] > 0.0
            use_log = gate & (kidx == 3)
            affine = gate & (kidx != 3)
            a_sel = jnp.where(kidx == 0, a0, jnp.where(kidx == 1, a1, a2))
            b_sel = jnp.where(kidx == 0, b0, jnp.where(kidx == 1, b1, 0.0))
            acc_ref[4:5, :] = jnp.where(affine, a_sel, 1.0)
            acc_ref[5:6, :] = jnp.where(affine, b_sel, 0.0)
            acc_ref[6:7, :] = jnp.where(use_log, 1.0, 0.0)

    @pl.when(p == 1)
    def _apply_phase():
        a = acc_ref[4:5, :]
        b = acc_ref[5:6, :]
        use_log = acc_ref[6:7, :] > 0.5
        x = xs_ref[pl.ds(i * rb, rb), :]
        lin = x * a + b
        o_ref[...] = jnp.where(use_log, _signed_log1p(x), lin)


def kernel(X, step_prob_logits, tf_prob_logits, is_train, max_only):
    B, F = X.shape
    K = tf_prob_logits.shape[1]
    sl = step_prob_logits.reshape(1, F)
    tl = tf_prob_logits.T
    nb = 2
    rb = B // nb

    return pl.pallas_call(
        functools.partial(_body, nb=nb, total_rows=B),
        grid=(2, nb),
        in_specs=[
            pl.BlockSpec((rb, F), lambda p, i: (i * (1 - p) + (nb - 1) * p, 0)),
            pl.BlockSpec((1, F), lambda p, i: (0, 0)),
            pl.BlockSpec((K, F), lambda p, i: (0, 0)),
        ],
        out_specs=pl.BlockSpec((rb, F), lambda p, i: (p * i, 0)),
        out_shape=jax.ShapeDtypeStruct((B, F), X.dtype),
        scratch_shapes=[
            pltpu.VMEM((8, F), jnp.float32),
            pltpu.VMEM((B, F), jnp.float32),
        ],
    )(X, sl, tl)
